# baseline (speedup 1.0000x reference)
import dataclasses
import functools

import jax
import jax.numpy as jnp
from jax import lax
from jax.experimental import pallas as pl
from jax.experimental.pallas import tpu as pltpu
from jax.experimental.pallas import tpu_sc as plsc

NC, NS, LANES = 2, 16, 16
NW = NC * NS


def _sc_compiler_params(tc_tiling):
    cp = pltpu.CompilerParams(use_tc_tiling_on_sc=tc_tiling)
    if "needs_layout_passes" in pltpu.CompilerParams.__dataclass_fields__:
        cp = dataclasses.replace(cp, needs_layout_passes=False)
    return cp


CW = 256


def _sc_pack_table(tok_t, tail2):
    E, V = tok_t.shape
    n_full = V // CW
    n_main = n_full // NW
    n_extra = n_full - n_main * NW
    n_tail = tail2.shape[0]
    IW = CW + 5
    OWP = 2 * E + 1

    mesh = plsc.VectorSubcoreMesh(core_axis_name="c", subcore_axis_name="s")

    @functools.partial(
        pl.kernel,
        out_type=jax.ShapeDtypeStruct((V // 2, 2 * E), jnp.float32),
        mesh=mesh,
        compiler_params=_sc_compiler_params(True),
        scratch_types=[
            pltpu.VMEM((2, E, IW), jnp.float32),
            pltpu.VMEM((2, CW // 2, OWP), jnp.float32),
            pltpu.SemaphoreType.DMA((2,)),
            pltpu.SemaphoreType.DMA((2,)),
        ],
    )
    def k1(tokt_hbm, tail2_hbm, tab2_hbm, inb_v, outb_v, isem, osem):
        wid = lax.axis_index("s") * NC + lax.axis_index("c")
        iota = lax.iota(jnp.int32, LANES)

        @pl.when(wid == NW - 1)
        def _():
            pltpu.sync_copy(tail2_hbm,
                            tab2_hbm.at[pl.ds(V // 2 - n_tail, n_tail)])

        def cid_of(i):
            return wid + NW * i

        def start_in(i, j):
            pltpu.async_copy(tokt_hbm.at[:, pl.ds(cid_of(i) * CW, CW)],
                             inb_v.at[j, :, pl.ds(0, CW)], isem.at[j])

        def wait_in(i, j):
            pltpu.make_async_copy(
                tokt_hbm.at[:, pl.ds(cid_of(i) * CW, CW)],
                inb_v.at[j, :, pl.ds(0, CW)], isem.at[j]).wait()

        def start_out(i, j):
            pltpu.async_copy(
                outb_v.at[j, :, pl.ds(0, 2 * E)],
                tab2_hbm.at[pl.ds(cid_of(i) * (CW // 2), CW // 2)],
                osem.at[j])

        def wait_out(i, j):
            pltpu.make_async_copy(
                outb_v.at[j, :, pl.ds(0, 2 * E)],
                tab2_hbm.at[pl.ds(cid_of(i) * (CW // 2), CW // 2)],
                osem.at[j]).wait()

        def transpose(j):
            @plsc.parallel_loop(0, CW, step=1, unroll=2)
            def _(v):
                half = (v & 1) * E
                row = v >> 1
                vv = jnp.full((LANES,), v, jnp.int32)
                for c in range(E // LANES):
                    val = plsc.load_gather(
                        inb_v.at[j], [iota + c * LANES, vv])
                    outb_v[j, row, pl.ds(half + c * LANES, LANES)] = val

        start_in(0, 0)
        start_in(1, 1)

        def step(i, j, prefetch, outwait):
            wait_in(i, j)
            if outwait:
                wait_out(i - 2, j)
            transpose(j)
            start_out(i, j)
            if prefetch:
                start_in(i + 2, j)

        step(0, 0, True, False)
        step(1, 1, True, False)

        @pl.loop(2, n_main - 2, step=2)
        def _(i0):
            step(i0, 0, True, True)
            step(i0 + 1, 1, True, True)

        step(n_main - 2, 0, False, True)
        step(n_main - 1, 1, False, True)
        wait_out(n_main - 2, 0)
        wait_out(n_main - 1, 1)

        if n_extra:
            @pl.when(wid < n_extra)
            def _():
                i = n_main
                start_in(i, 0)
                wait_in(i, 0)
                transpose(0)
                start_out(i, 0)
                wait_out(i, 0)

    return k1(tok_t, tail2)


def _sc_embed(x_t, tab, pos_et):
    L, B = x_t.shape
    V, E = tab.shape
    BW = B // NW
    ET, BT = E // 8, B // 128
    OWP = 129
    NBUF = 3
    NOB = 2

    mesh = plsc.VectorSubcoreMesh(core_axis_name="c", subcore_axis_name="s")

    @functools.partial(
        pl.kernel,
        out_type=jax.ShapeDtypeStruct((L, ET, BT, 8, 128), jnp.float32),
        mesh=mesh,
        compiler_params=_sc_compiler_params(False),
        scratch_types=[
            pltpu.VMEM((L, BW), jnp.int32),
            pltpu.VMEM((NBUF, BW, E), jnp.float32),
            pltpu.VMEM((NOB, ET, 8, OWP), jnp.float32),
            pltpu.VMEM((E, L), jnp.float32),
            pltpu.SemaphoreType.DMA((NBUF,)),
            pltpu.SemaphoreType.DMA((NOB,)),
        ],
    )
    def k(xt_hbm, tab_hbm, pos_hbm, out_hbm, idxs_v, rows_v, outb_v, pos_v,
          gsem, osem):
        wid = lax.axis_index("s") * NC + lax.axis_index("c")
        bbase = wid * BW
        iota = lax.iota(jnp.int32, LANES)

        pltpu.sync_copy(pos_hbm, pos_v)
        pltpu.sync_copy(xt_hbm.at[:, pl.ds(bbase, BW)], idxs_v)

        def start_gather(t, jb):
            pltpu.async_copy(
                tab_hbm.at[idxs_v.at[t]], rows_v.at[jb], gsem.at[jb])

        def wait_gather(t, jb):
            pltpu.make_async_copy(
                tab_hbm.at[idxs_v.at[t]], rows_v.at[jb], gsem.at[jb]).wait()

        def start_out(t, q):
            pltpu.async_copy(
                outb_v.at[q, :, :, pl.ds(0, 128)],
                out_hbm.at[t, :, wid], osem.at[q])

        def wait_out(t, q):
            pltpu.make_async_copy(
                outb_v.at[q, :, :, pl.ds(0, 128)],
                out_hbm.at[t, :, wid], osem.at[q]).wait()

        def transpose_add(t, jb, q):
            lvec = jnp.full((LANES,), t, jnp.int32)
            pos_c = [
                plsc.load_gather(pos_v, [iota + c * LANES, lvec])
                for c in range(E // LANES)
            ]
            e_hi = [(iota + c * LANES) >> 3 for c in range(E // LANES)]
            e_lo = [(iota + c * LANES) & 7 for c in range(E // LANES)]

            @plsc.parallel_loop(0, BW, step=1, unroll=2)
            def _(b):
                bvec = jnp.full((LANES,), b, jnp.int32)
                for c in range(E // LANES):
                    val = rows_v[jb, b, pl.ds(c * LANES, LANES)]
                    plsc.store_scatter(
                        outb_v.at[q], [e_hi[c], e_lo[c], bvec],
                        val + pos_c[c])

        for t in range(NBUF):
            start_gather(t, t)

        def step(t, jb, q, prefetch, outwait):
            wait_gather(t, jb)
            if outwait:
                wait_out(t - NOB, q)
            transpose_add(t, jb, q)
            start_out(t, q)
            if prefetch:
                start_gather(t + NBUF, jb)

        for t in range(NOB):
            step(t, t % NBUF, t % NOB, True, False)

        STEP = NBUF * NOB
        body_lo = NOB
        n_mid = ((L - NBUF - body_lo) // STEP) * STEP
        mid_hi = body_lo + n_mid

        @pl.loop(body_lo, mid_hi, step=STEP)
        def _(t0):
            for j in range(STEP):
                step(t0 + j, (body_lo + j) % NBUF, j % NOB, True, True)

        for t in range(mid_hi, L):
            step(t, t % NBUF, t % NOB, t + NBUF < L, True)

        for t in range(L - NOB, L):
            wait_out(t, t % NOB)

    return k(x_t, tab, pos_et)


@jax.jit
def kernel(x, token_table, pos_table):
    B, L = x.shape
    V, E = token_table.shape
    x_t = x.T.astype(jnp.int32)
    pos_et = pos_table[:L].T.astype(jnp.float32)
    tok_t = token_table.T
    vt = (V // CW) * CW
    tail2 = token_table[vt:].reshape((V - vt) // 2, 2 * E)
    tab2 = _sc_pack_table(tok_t, tail2)
    tab = jnp.reshape(tab2, (V, E))
    out5 = _sc_embed(x_t, tab, pos_et)
    out = jnp.transpose(out5, (2, 4, 0, 1, 3))
    return jnp.reshape(out, (B, L, E))


# speedup vs baseline: 1.2693x; 1.2693x over previous
import dataclasses
import functools

import jax
import jax.numpy as jnp
from jax import lax
from jax.experimental import pallas as pl
from jax.experimental.pallas import tpu as pltpu
from jax.experimental.pallas import tpu_sc as plsc

NC, NS, LANES = 2, 16, 16
NW = NC * NS


def _sc_compiler_params(tc_tiling):
    cp = pltpu.CompilerParams(use_tc_tiling_on_sc=tc_tiling)
    if "needs_layout_passes" in pltpu.CompilerParams.__dataclass_fields__:
        cp = dataclasses.replace(cp, needs_layout_passes=False)
    return cp


def _sc_embed(x_t, tab, pos_et):
    L, B = x_t.shape
    V, E = tab.shape
    BW = B // NW
    ET, BT = E // 8, B // 128
    OWP = 129
    NBUF = 3
    NOB = 2

    mesh = plsc.VectorSubcoreMesh(core_axis_name="c", subcore_axis_name="s")

    @functools.partial(
        pl.kernel,
        out_type=jax.ShapeDtypeStruct((L, ET, BT, 8, 128), jnp.float32),
        mesh=mesh,
        compiler_params=_sc_compiler_params(False),
        scratch_types=[
            pltpu.VMEM((L, BW), jnp.int32),
            pltpu.VMEM((NBUF, BW, E), jnp.float32),
            pltpu.VMEM((NOB, ET, 8, OWP), jnp.float32),
            pltpu.VMEM((E, L), jnp.float32),
            pltpu.SemaphoreType.DMA((NBUF,)),
            pltpu.SemaphoreType.DMA((NOB,)),
        ],
    )
    def k(xt_hbm, tab_hbm, pos_hbm, out_hbm, idxs_v, rows_v, outb_v, pos_v,
          gsem, osem):
        wid = lax.axis_index("s") * NC + lax.axis_index("c")
        bbase = wid * BW
        iota = lax.iota(jnp.int32, LANES)

        pltpu.sync_copy(pos_hbm, pos_v)
        pltpu.sync_copy(xt_hbm.at[:, pl.ds(bbase, BW)], idxs_v)

        def start_gather(t, jb):
            pltpu.async_copy(
                tab_hbm.at[idxs_v.at[t]], rows_v.at[jb], gsem.at[jb])

        def wait_gather(t, jb):
            pltpu.make_async_copy(
                tab_hbm.at[idxs_v.at[t]], rows_v.at[jb], gsem.at[jb]).wait()

        def start_out(t, q):
            pltpu.async_copy(
                outb_v.at[q, :, :, pl.ds(0, 128)],
                out_hbm.at[t, :, wid], osem.at[q])

        def wait_out(t, q):
            pltpu.make_async_copy(
                outb_v.at[q, :, :, pl.ds(0, 128)],
                out_hbm.at[t, :, wid], osem.at[q]).wait()

        def transpose_add(t, jb, q):
            lvec = jnp.full((LANES,), t, jnp.int32)
            pos_c = [
                plsc.load_gather(pos_v, [iota + c * LANES, lvec])
                for c in range(E // LANES)
            ]
            e_hi = [(iota + c * LANES) >> 3 for c in range(E // LANES)]
            e_lo = [(iota + c * LANES) & 7 for c in range(E // LANES)]

            @plsc.parallel_loop(0, BW, step=1, unroll=2)
            def _(b):
                bvec = jnp.full((LANES,), b, jnp.int32)
                for c in range(E // LANES):
                    val = rows_v[jb, b, pl.ds(c * LANES, LANES)]
                    plsc.store_scatter(
                        outb_v.at[q], [e_hi[c], e_lo[c], bvec],
                        val + pos_c[c])

        for t in range(NBUF):
            start_gather(t, t)

        def step(t, jb, q, prefetch, outwait):
            wait_gather(t, jb)
            if outwait:
                wait_out(t - NOB, q)
            transpose_add(t, jb, q)
            start_out(t, q)
            if prefetch:
                start_gather(t + NBUF, jb)

        for t in range(NOB):
            step(t, t % NBUF, t % NOB, True, False)

        STEP = NBUF * NOB
        body_lo = NOB
        n_mid = ((L - NBUF - body_lo) // STEP) * STEP
        mid_hi = body_lo + n_mid

        @pl.loop(body_lo, mid_hi, step=STEP)
        def _(t0):
            for j in range(STEP):
                step(t0 + j, (body_lo + j) % NBUF, j % NOB, True, True)

        for t in range(mid_hi, L):
            step(t, t % NBUF, t % NOB, t + NBUF < L, True)

        for t in range(L - NOB, L):
            wait_out(t, t % NOB)

    return k(x_t, tab, pos_et)


@jax.jit
def kernel(x, token_table, pos_table):
    B, L = x.shape
    V, E = token_table.shape
    x_t = x.T.astype(jnp.int32)
    pos_et = pos_table[:L].T.astype(jnp.float32)
    tab2 = lax.optimization_barrier(
        jnp.reshape(token_table, (V // 2, 2 * E)))
    tab = jnp.reshape(tab2, (V, E))
    out5 = _sc_embed(x_t, tab, pos_et)
    out = jnp.transpose(out5, (2, 4, 0, 1, 3))
    return jnp.reshape(out, (B, L, E))
